# Initial kernel scaffold; baseline (speedup 1.0000x reference)
import jax
import jax.numpy as jnp
from jax.experimental import pallas as pl


def kernel(x):
    raise NotImplementedError("write your pallas kernel here")


# speedup vs baseline: 3.0881x; 3.0881x over previous
import jax
import jax.numpy as jnp
from jax.experimental import pallas as pl

ROWS = 4096
COLS = 16384
CHUNK = 128
NCHUNK = COLS // CHUNK
BR = 64


def _tri(n, strict):
    i = jax.lax.broadcasted_iota(jnp.int32, (n, n), 0)
    j = jax.lax.broadcasted_iota(jnp.int32, (n, n), 1)
    return ((i < j) if strict else (i <= j)).astype(jnp.float32)


def _cumsum_block(x_ref, o_ref):
    xb = x_ref[...]
    u_incl = _tri(CHUNK, strict=False)
    u_excl = _tri(NCHUNK, strict=True)
    y = jax.lax.dot_general(
        xb, u_incl,
        dimension_numbers=(((2,), (0,)), ((), ())),
        precision=jax.lax.Precision.HIGHEST,
        preferred_element_type=jnp.float32,
    )
    totals = y[:, :, CHUNK - 1]
    carry = jax.lax.dot_general(
        totals, u_excl,
        dimension_numbers=(((1,), (0,)), ((), ())),
        precision=jax.lax.Precision.HIGHEST,
        preferred_element_type=jnp.float32,
    )
    o_ref[...] = y + carry[:, :, None]


@jax.jit
def kernel(x):
    x3 = x.reshape(ROWS, NCHUNK, CHUNK)
    out = pl.pallas_call(
        _cumsum_block,
        grid=(ROWS // BR,),
        in_specs=[pl.BlockSpec((BR, NCHUNK, CHUNK), lambda i: (i, 0, 0))],
        out_specs=pl.BlockSpec((BR, NCHUNK, CHUNK), lambda i: (i, 0, 0)),
        out_shape=jax.ShapeDtypeStruct((ROWS, NCHUNK, CHUNK), jnp.float32),
    )(x3)
    return out.reshape(ROWS, COLS)


# speedup vs baseline: 6.2550x; 2.0255x over previous
import jax
import jax.numpy as jnp
from jax.experimental import pallas as pl

ROWS = 4096
COLS = 16384
CHUNK = 128
NCHUNK = COLS // CHUNK
BR = 64


def _tri(n, strict):
    i = jax.lax.broadcasted_iota(jnp.int32, (n, n), 0)
    j = jax.lax.broadcasted_iota(jnp.int32, (n, n), 1)
    return ((i < j) if strict else (i <= j)).astype(jnp.float32)


def _cumsum_block(x_ref, o_ref):
    xb = x_ref[...].reshape(BR, NCHUNK, CHUNK)
    u_incl = _tri(CHUNK, strict=False)
    u_excl = _tri(NCHUNK, strict=True)
    y = jax.lax.dot_general(
        xb, u_incl,
        dimension_numbers=(((2,), (0,)), ((), ())),
        precision=jax.lax.Precision.HIGHEST,
        preferred_element_type=jnp.float32,
    )
    totals = y[:, :, CHUNK - 1]
    carry = jax.lax.dot_general(
        totals, u_excl,
        dimension_numbers=(((1,), (0,)), ((), ())),
        precision=jax.lax.Precision.HIGHEST,
        preferred_element_type=jnp.float32,
    )
    o_ref[...] = (y + carry[:, :, None]).reshape(BR, COLS)


@jax.jit
def kernel(x):
    return pl.pallas_call(
        _cumsum_block,
        grid=(ROWS // BR,),
        in_specs=[pl.BlockSpec((BR, COLS), lambda i: (i, 0))],
        out_specs=pl.BlockSpec((BR, COLS), lambda i: (i, 0)),
        out_shape=jax.ShapeDtypeStruct((ROWS, COLS), jnp.float32),
    )(x)


# speedup vs baseline: 10.5520x; 1.6870x over previous
import jax
import jax.numpy as jnp
from jax.experimental import pallas as pl

ROWS = 4096
COLS = 16384
CHUNK = 128
NCHUNK = COLS // CHUNK
BR = 64


def _tri(n, strict):
    i = jax.lax.broadcasted_iota(jnp.int32, (n, n), 0)
    j = jax.lax.broadcasted_iota(jnp.int32, (n, n), 1)
    return ((i < j) if strict else (i <= j)).astype(jnp.float32)


def _cumsum_block(x_ref, o_ref):
    xb = x_ref[...].reshape(BR, NCHUNK, CHUNK)
    u_incl = _tri(CHUNK, strict=False)
    u_excl = _tri(NCHUNK, strict=True)
    y = jax.lax.dot_general(
        xb, u_incl,
        dimension_numbers=(((2,), (0,)), ((), ())),
        precision=jax.lax.Precision.DEFAULT,
        preferred_element_type=jnp.float32,
    )
    totals = y[:, :, CHUNK - 1]
    carry = jax.lax.dot_general(
        totals, u_excl,
        dimension_numbers=(((1,), (0,)), ((), ())),
        precision=jax.lax.Precision.DEFAULT,
        preferred_element_type=jnp.float32,
    )
    o_ref[...] = (y + carry[:, :, None]).reshape(BR, COLS)


@jax.jit
def kernel(x):
    return pl.pallas_call(
        _cumsum_block,
        grid=(ROWS // BR,),
        in_specs=[pl.BlockSpec((BR, COLS), lambda i: (i, 0))],
        out_specs=pl.BlockSpec((BR, COLS), lambda i: (i, 0)),
        out_shape=jax.ShapeDtypeStruct((ROWS, COLS), jnp.float32),
    )(x)
